# baseline (speedup 1.0000x reference)
import functools

import jax
import jax.numpy as jnp
from jax import lax
from jax.experimental import pallas as pl
from jax.experimental.pallas import tpu as pltpu
from jax.experimental.pallas import tpu_sc as plsc

B, T, K, D = 8, 576, 8192, 256
R = B * T
NT = 3328
NS = R - NT
NW = 32
RPA = NS // NW
RPB = NT // NW
L = 16
U = 8
CH = K // L
RB = 128
BIG = 2**30


def _sc_argmin_gather_body(x_hbm, tab_hbm, out_hbm, buf, idx_v, rows_v,
                           sem0, sem1, sem2, sem3, semg):
    wid = lax.axis_index("s") * 2 + lax.axis_index("c")
    base = NT + wid * RPA
    iota = lax.iota(jnp.int32, L)
    sems = (sem0, sem1, sem2, sem3)

    def start_pair(p, b):
        @pl.when(p < RPA // 2)
        def _():
            pltpu.async_copy(x_hbm.at[pl.ds(base + 2 * p, 2)], buf.at[b],
                             sems[b])

    def wait_pair(b):
        pltpu.make_async_copy(x_hbm.at[pl.ds(base, 2)], buf.at[b],
                              sems[b]).wait()

    for b in range(4):
        start_pair(b, b)

    def argmin_row(b, rr):
        def step(g, carry):
            minvs, minis = carry
            gsplat = jnp.full((L,), g, jnp.int32)
            off = g * (U * L)
            new_v, new_i = [], []
            for j in range(U):
                v = buf[b, rr, pl.ds(off + j * L, L)]
                c = v < minvs[j]
                new_v.append(jnp.where(c, v, minvs[j]))
                new_i.append(jnp.where(c, gsplat, minis[j]))
            return tuple(new_v), tuple(new_i)

        iv = tuple(jnp.full((L,), jnp.inf, jnp.float32) for _ in range(U))
        ii = tuple(jnp.zeros((L,), jnp.int32) for _ in range(U))
        minvs, minis = lax.fori_loop(0, CH // U, step, (iv, ii))
        mv = minvs[0]
        for j in range(1, U):
            mv = jnp.minimum(mv, minvs[j])
        m = jnp.min(mv)
        cand = jnp.full((L,), BIG, jnp.int32)
        for j in range(U):
            cj = jnp.where(
                minvs[j] == m,
                (minis[j] * U + j) * L + iota,
                jnp.full((L,), BIG, jnp.int32),
            )
            cand = jnp.minimum(cand, cj)
        return jnp.min(cand)

    def ring_step(o, _):
        for bslot in range(4):
            p = o * 4 + bslot
            wait_pair(bslot)
            for rr in range(2):
                pos = argmin_row(bslot, rr)
                plsc.store_scatter(
                    idx_v,
                    [jnp.full((L,), 2 * p + rr, jnp.int32)],
                    jnp.full((L,), pos, jnp.int32),
                    mask=iota == 0,
                )
            start_pair(p + 4, bslot)
        return 0

    lax.fori_loop(0, RPA // 8, ring_step, 0)

    pltpu.async_copy(tab_hbm.at[idx_v], rows_v, semg).wait()
    pltpu.sync_copy(rows_v, out_hbm.at[pl.ds(base, RPA)])


def _sc_gather_body(pos_hbm, tab_hbm, out_hbm, idx_v, rows_v, sem):
    wid = lax.axis_index("s") * 2 + lax.axis_index("c")
    base = wid * RPB
    pltpu.sync_copy(pos_hbm.at[pl.ds(base, RPB)], idx_v)
    pltpu.async_copy(tab_hbm.at[idx_v], rows_v, sem).wait()
    pltpu.sync_copy(rows_v, out_hbm.at[pl.ds(base, RPB)])


def _tc_argmin_body(x_ref, o_ref):
    x = x_ref[...]
    m = jnp.min(x, axis=1, keepdims=True)
    ii = lax.broadcasted_iota(jnp.int32, (RB, K), 1)
    cand = jnp.where(x == m, ii, BIG)
    o_ref[...] = jnp.min(cand, axis=1).reshape(1, 1, RB)


@jax.jit
def _snap(x2d, tab):
    mesh = plsc.VectorSubcoreMesh(core_axis_name="c", subcore_axis_name="s")
    out_ref = jax.new_ref(jnp.zeros((R, D), jnp.float32))

    pl.kernel(
        _sc_argmin_gather_body,
        out_type=(),
        mesh=mesh,
        compiler_params=pltpu.CompilerParams(needs_layout_passes=False),
        scratch_types=[
            pltpu.VMEM((4, 2, K), jnp.float32),
            pltpu.VMEM((RPA,), jnp.int32),
            pltpu.VMEM((RPA, D), jnp.float32),
            pltpu.SemaphoreType.DMA,
            pltpu.SemaphoreType.DMA,
            pltpu.SemaphoreType.DMA,
            pltpu.SemaphoreType.DMA,
            pltpu.SemaphoreType.DMA,
        ],
    )(x2d, tab, out_ref)

    pos_tc = pl.pallas_call(
        _tc_argmin_body,
        grid=(NT // RB,),
        in_specs=[pl.BlockSpec((RB, K), lambda i: (i, 0))],
        out_specs=pl.BlockSpec((1, 1, RB), lambda i: (i, 0, 0)),
        out_shape=jax.ShapeDtypeStruct((NT // RB, 1, RB), jnp.int32),
    )(x2d).reshape(NT)

    pl.kernel(
        _sc_gather_body,
        out_type=(),
        mesh=mesh,
        compiler_params=pltpu.CompilerParams(needs_layout_passes=False),
        scratch_types=[
            pltpu.VMEM((RPB,), jnp.int32),
            pltpu.VMEM((RPB, D), jnp.float32),
            pltpu.SemaphoreType.DMA,
        ],
    )(pos_tc, tab, out_ref)

    return out_ref[...]


def kernel(inputs, reference_table):
    out = _snap(inputs.reshape(R, K), reference_table)
    return out.reshape(B, T, D)


# speedup vs baseline: 1.0858x; 1.0858x over previous
import functools

import jax
import jax.numpy as jnp
from jax import lax
from jax.experimental import pallas as pl
from jax.experimental.pallas import tpu as pltpu
from jax.experimental.pallas import tpu_sc as plsc

B, T, K, D = 8, 576, 8192, 256
R = B * T
NT = 3328
NS = R - NT
NW = 32
RPA = NS // NW
RPB = NT // NW
L = 16
U = 8
CH = K // L
RB = 256
BIG = 2**30


def _sc_argmin_gather_body(x_hbm, tab_hbm, out_hbm, buf, idx_v, rows_v,
                           sem0, sem1, semg):
    wid = lax.axis_index("s") * 2 + lax.axis_index("c")
    base = NT + wid * RPA
    iota = lax.iota(jnp.int32, L)
    sems = (sem0, sem1)

    def start_row(r, b):
        @pl.when(r < RPA)
        def _():
            pltpu.async_copy(x_hbm.at[base + r], buf.at[b], sems[b])

    def wait_row(b):
        pltpu.make_async_copy(x_hbm.at[base], buf.at[b], sems[b]).wait()

    start_row(0, 0)
    start_row(1, 1)

    def argmin_row(b):
        def step(g, carry):
            minvs, minis = carry
            gsplat = jnp.full((L,), g, jnp.int32)
            off = g * (U * L)
            new_v, new_i = [], []
            for j in range(U):
                v = buf[b, pl.ds(off + j * L, L)]
                c = v < minvs[j]
                new_v.append(jnp.where(c, v, minvs[j]))
                new_i.append(jnp.where(c, gsplat, minis[j]))
            return tuple(new_v), tuple(new_i)

        iv = tuple(jnp.full((L,), jnp.inf, jnp.float32) for _ in range(U))
        ii = tuple(jnp.zeros((L,), jnp.int32) for _ in range(U))
        minvs, minis = lax.fori_loop(0, CH // U, step, (iv, ii))
        mv = minvs[0]
        for j in range(1, U):
            mv = jnp.minimum(mv, minvs[j])
        m = jnp.min(mv)
        cand = jnp.full((L,), BIG, jnp.int32)
        for j in range(U):
            cj = jnp.where(
                minvs[j] == m,
                (minis[j] * U + j) * L + iota,
                jnp.full((L,), BIG, jnp.int32),
            )
            cand = jnp.minimum(cand, cj)
        return jnp.min(cand)

    def row_pair(r0, _):
        for bslot in range(2):
            r = r0 * 2 + bslot
            wait_row(bslot)
            pos = argmin_row(bslot)
            start_row(r + 2, bslot)
            plsc.store_scatter(
                idx_v,
                [jnp.full((L,), r, jnp.int32)],
                jnp.full((L,), pos, jnp.int32),
                mask=iota == 0,
            )
        return 0

    lax.fori_loop(0, RPA // 2, row_pair, 0)

    pltpu.async_copy(tab_hbm.at[idx_v], rows_v, semg).wait()
    pltpu.sync_copy(rows_v, out_hbm.at[pl.ds(base, RPA)])


def _sc_gather_body(pos_hbm, tab_hbm, out_hbm, idx_v, rows_v, sem):
    wid = lax.axis_index("s") * 2 + lax.axis_index("c")
    base = wid * RPB
    pltpu.sync_copy(pos_hbm.at[pl.ds(base, RPB)], idx_v)
    pltpu.async_copy(tab_hbm.at[idx_v], rows_v, sem).wait()
    pltpu.sync_copy(rows_v, out_hbm.at[pl.ds(base, RPB)])


def _tc_argmin_body(x_ref, o_ref):
    x = x_ref[...]
    m = jnp.min(x, axis=1, keepdims=True)
    ii = lax.broadcasted_iota(jnp.int32, (RB, K), 1)
    cand = jnp.where(x == m, ii, BIG)
    o_ref[...] = jnp.min(cand, axis=1).reshape(1, 1, RB)


@jax.jit
def _snap(x2d, tab):
    mesh = plsc.VectorSubcoreMesh(core_axis_name="c", subcore_axis_name="s")
    out_ref = jax.new_ref(jnp.zeros((R, D), jnp.float32))

    pl.kernel(
        _sc_argmin_gather_body,
        out_type=(),
        mesh=mesh,
        compiler_params=pltpu.CompilerParams(needs_layout_passes=False),
        scratch_types=[
            pltpu.VMEM((2, K), jnp.float32),
            pltpu.VMEM((RPA,), jnp.int32),
            pltpu.VMEM((RPA, D), jnp.float32),
            pltpu.SemaphoreType.DMA,
            pltpu.SemaphoreType.DMA,
            pltpu.SemaphoreType.DMA,
        ],
    )(x2d, tab, out_ref)

    pos_tc = pl.pallas_call(
        _tc_argmin_body,
        grid=(NT // RB,),
        in_specs=[pl.BlockSpec((RB, K), lambda i: (i, 0))],
        out_specs=pl.BlockSpec((1, 1, RB), lambda i: (i, 0, 0)),
        out_shape=jax.ShapeDtypeStruct((NT // RB, 1, RB), jnp.int32),
    )(x2d).reshape(NT)

    pl.kernel(
        _sc_gather_body,
        out_type=(),
        mesh=mesh,
        compiler_params=pltpu.CompilerParams(needs_layout_passes=False),
        scratch_types=[
            pltpu.VMEM((RPB,), jnp.int32),
            pltpu.VMEM((RPB, D), jnp.float32),
            pltpu.SemaphoreType.DMA,
        ],
    )(pos_tc, tab, out_ref)

    return out_ref[...]


def kernel(inputs, reference_table):
    out = _snap(inputs.reshape(R, K), reference_table)
    return out.reshape(B, T, D)
